# initial kernel scaffold (unmeasured)
import jax
import jax.numpy as jnp
from jax import lax
from jax.experimental import pallas as pl
from jax.experimental.pallas import tpu as pltpu

N_DEV = 4
SQ = 2048
HQ = 8
DH = 128
DM = HQ * DH
BLK = 64
SCALE = 0.08838834764831843
QCHUNK = 512
N_QC = SQ // QCHUNK


def kernel(x, Wq, K_ext, V_ext, Wo):
    def body(x_ref, wq_ref, k_ref, v_ref, wo_ref, out_ref, ctx_ref,
             send_sem, recv_sem):
        my = lax.axis_index("i")
        right = (my + 1) % N_DEV

        @pl.when(my == 0)
        def _compute_ctx():
            q = jnp.dot(x_ref[0], wq_ref[...],
                        preferred_element_type=jnp.float32)
            for h in range(HQ):
                kh = k_ref[0, :, h, :]
                vh = v_ref[0, :, h, :]
                qh = q[:, h * DH:(h + 1) * DH]
                for qc in range(N_QC):
                    r0 = qc * QCHUNK
                    kn = (qc + 1) * QCHUNK
                    s = lax.dot_general(
                        qh[r0:r0 + QCHUNK], kh[:kn],
                        (((1,), (1,)), ((), ())),
                        preferred_element_type=jnp.float32,
                    ) * SCALE
                    rows = r0 + lax.broadcasted_iota(
                        jnp.int32, (QCHUNK, kn), 0)
                    cols = lax.broadcasted_iota(jnp.int32, (QCHUNK, kn), 1)
                    s = jnp.where((cols // BLK) <= (rows // BLK), s, -1e9)
                    m = jnp.max(s, axis=1, keepdims=True)
                    w = jnp.exp(s - m)
                    w = w / jnp.sum(w, axis=1, keepdims=True)
                    ctx_ref[r0:r0 + QCHUNK, h * DH:(h + 1) * DH] = jnp.dot(
                        w, vh[:kn], preferred_element_type=jnp.float32)

        copy = pltpu.make_async_remote_copy(
            src_ref=ctx_ref,
            dst_ref=ctx_ref,
            send_sem=send_sem,
            recv_sem=recv_sem,
            device_id=(right,),
            device_id_type=pl.DeviceIdType.MESH,
        )

        @pl.when(my > 0)
        def _recv():
            copy.wait_recv()

        @pl.when(my < N_DEV - 1)
        def _send():
            copy.start()
            copy.wait_send()

        out_ref[0] = jnp.dot(ctx_ref[...], wo_ref[...],
                             preferred_element_type=jnp.float32)

    return pl.pallas_call(
        body,
        out_shape=jax.ShapeDtypeStruct((1, SQ, DM), jnp.float32),
        in_specs=[pl.BlockSpec(memory_space=pltpu.VMEM)] * 5,
        out_specs=pl.BlockSpec(memory_space=pltpu.VMEM),
        scratch_shapes=[
            pltpu.VMEM((SQ, DM), jnp.float32),
            pltpu.SemaphoreType.DMA,
            pltpu.SemaphoreType.DMA,
        ],
    )(x, Wq, K_ext, V_ext, Wo)


# baseline (device time: 368501 ns/iter reference)
import jax
import jax.numpy as jnp
from jax import lax
from jax.experimental import pallas as pl
from jax.experimental.pallas import tpu as pltpu

N_DEV = 4
SQ = 2048
SKV = 2048
HQ = 8
DH = 128
DM = HQ * DH
BLK = 64
SCALE = 0.08838834764831843
QC = 256
N_AQC = SQ // QC
OC = 512
N_OC = SQ // OC


def kernel(x, Wq, K_ext, V_ext, Wo):
    def body(x_ref, wq_ref, k_ref, v_ref, wo_ref, out_ref,
             ctx_ref, xc_ref, kh_ref, vh_ref, qh_ref, oc_ref,
             dma_sem, ksem, vsem, send_sem, recv_sem):
        my = lax.axis_index("i")
        right = (my + 1) % N_DEV

        @pl.when(my == 0)
        def _attn():
            for c in range(N_OC):
                r0 = c * OC
                cp = pltpu.make_async_copy(
                    x_ref.at[0, pl.ds(r0, OC)], xc_ref, dma_sem)
                cp.start()
                cp.wait()
                oc_ref[...] = jnp.dot(xc_ref[...], wq_ref[...],
                                      preferred_element_type=jnp.float32)
                cp2 = pltpu.make_async_copy(
                    oc_ref, out_ref.at[0, pl.ds(r0, OC)], dma_sem)
                cp2.start()
                cp2.wait()

            for h in range(HQ):
                ck = pltpu.make_async_copy(k_ref.at[0, :, h, :], kh_ref, ksem)
                cv = pltpu.make_async_copy(v_ref.at[0, :, h, :], vh_ref, vsem)
                cq = pltpu.make_async_copy(
                    out_ref.at[0, :, pl.ds(h * DH, DH)], qh_ref, dma_sem)
                ck.start()
                cv.start()
                cq.start()
                ck.wait()
                cv.wait()
                cq.wait()
                for qc in range(N_AQC):
                    r0 = qc * QC
                    kn = r0 + QC
                    s = lax.dot_general(
                        qh_ref[r0:r0 + QC, :], kh_ref[:kn, :],
                        (((1,), (1,)), ((), ())),
                        preferred_element_type=jnp.float32,
                    ) * SCALE
                    rb = (r0 + lax.broadcasted_iota(
                        jnp.int32, (QC, 1), 0)) // BLK
                    cb = lax.broadcasted_iota(jnp.int32, (1, kn), 1) // BLK
                    s = jnp.where(cb <= rb, s, -1e9)
                    m = jnp.max(s, axis=1, keepdims=True)
                    w = jnp.exp(s - m)
                    w = w / jnp.sum(w, axis=1, keepdims=True)
                    ctx_ref[r0:r0 + QC, h * DH:(h + 1) * DH] = jnp.dot(
                        w, vh_ref[:kn, :], preferred_element_type=jnp.float32)

        copy = pltpu.make_async_remote_copy(
            src_ref=ctx_ref,
            dst_ref=ctx_ref,
            send_sem=send_sem,
            recv_sem=recv_sem,
            device_id=(right,),
            device_id_type=pl.DeviceIdType.MESH,
        )

        @pl.when(my > 0)
        def _recv():
            copy.wait_recv()

        @pl.when(my < N_DEV - 1)
        def _send():
            copy.start()
            copy.wait_send()

        for c in range(N_OC):
            r0 = c * OC
            oc_ref[...] = jnp.dot(ctx_ref[r0:r0 + OC, :], wo_ref[...],
                                  preferred_element_type=jnp.float32)
            cp = pltpu.make_async_copy(
                oc_ref, out_ref.at[0, pl.ds(r0, OC)], dma_sem)
            cp.start()
            cp.wait()

    return pl.pallas_call(
        body,
        out_shape=jax.ShapeDtypeStruct((1, SQ, DM), jnp.float32),
        in_specs=[
            pl.BlockSpec(memory_space=pl.ANY),
            pl.BlockSpec(memory_space=pltpu.VMEM),
            pl.BlockSpec(memory_space=pl.ANY),
            pl.BlockSpec(memory_space=pl.ANY),
            pl.BlockSpec(memory_space=pltpu.VMEM),
        ],
        out_specs=pl.BlockSpec(memory_space=pl.ANY),
        scratch_shapes=[
            pltpu.VMEM((SQ, DM), jnp.float32),
            pltpu.VMEM((OC, DM), jnp.float32),
            pltpu.VMEM((SKV, DH), jnp.float32),
            pltpu.VMEM((SKV, DH), jnp.float32),
            pltpu.VMEM((SQ, DH), jnp.float32),
            pltpu.VMEM((OC, DM), jnp.float32),
            pltpu.SemaphoreType.DMA,
            pltpu.SemaphoreType.DMA,
            pltpu.SemaphoreType.DMA,
            pltpu.SemaphoreType.DMA,
            pltpu.SemaphoreType.DMA,
        ],
    )(x, Wq, K_ext, V_ext, Wo)


# device time: 127451 ns/iter; 2.8913x vs baseline; 2.8913x over previous
import jax
import jax.numpy as jnp
from jax import lax
from jax.experimental import pallas as pl
from jax.experimental.pallas import tpu as pltpu

N_DEV = 4
SQ = 2048
SKV = 2048
HQ = 8
DH = 128
DM = HQ * DH
BLK = 64
SCALE = 0.08838834764831843
QC = 256
N_AQC = SQ // QC
OC = 512
N_OC = SQ // OC

EVENS = (0, 2, 4, 6)
ODDS = (1, 3, 5, 7)


def kernel(x, Wq, K_ext, V_ext, Wo):
    def body(x_ref, wq_ref, k_ref, v_ref, wo_ref, out_ref,
             ctx_ref, xc_ref, kh_ref, vh_ref, qh_ref, oc_ref,
             dma_sem, ksem, vsem, qsem, send_sems, recv_sems):
        my = lax.axis_index("i")

        def chunk_copy(h, tgt):
            return pltpu.make_async_remote_copy(
                src_ref=ctx_ref.at[h],
                dst_ref=ctx_ref.at[h],
                send_sem=send_sems.at[h],
                recv_sem=recv_sems.at[h],
                device_id=(tgt,),
                device_id_type=pl.DeviceIdType.MESH,
            )

        def head_loads(h):
            slot = h % 2
            return (
                pltpu.make_async_copy(
                    k_ref.at[0, :, h, :], kh_ref.at[slot], ksem.at[slot]),
                pltpu.make_async_copy(
                    v_ref.at[0, :, h, :], vh_ref.at[slot], vsem.at[slot]),
                pltpu.make_async_copy(
                    out_ref.at[0, :, pl.ds(h * DH, DH)], qh_ref.at[slot],
                    qsem.at[slot]),
            )

        @pl.when(my == 0)
        def _attn():
            for c in range(N_OC):
                r0 = c * OC
                cp = pltpu.make_async_copy(
                    x_ref.at[0, pl.ds(r0, OC)], xc_ref, dma_sem)
                cp.start()
                cp.wait()
                oc_ref[...] = jnp.dot(xc_ref[...], wq_ref[...],
                                      preferred_element_type=jnp.float32)
                cp2 = pltpu.make_async_copy(
                    oc_ref, out_ref.at[0, pl.ds(r0, OC)], dma_sem)
                cp2.start()
                cp2.wait()

            for op in head_loads(0):
                op.start()
            for h in range(HQ):
                if h + 1 < HQ:
                    for op in head_loads(h + 1):
                        op.start()
                for op in head_loads(h):
                    op.wait()
                slot = h % 2
                for qc in range(N_AQC):
                    r0 = qc * QC
                    kn = r0 + QC
                    s = lax.dot_general(
                        qh_ref[slot, r0:r0 + QC, :], kh_ref[slot, :kn, :],
                        (((1,), (1,)), ((), ())),
                        preferred_element_type=jnp.float32,
                    ) * SCALE
                    rb = (r0 + lax.broadcasted_iota(
                        jnp.int32, (QC, 1), 0)) // BLK
                    cb = lax.broadcasted_iota(jnp.int32, (1, kn), 1) // BLK
                    s = jnp.where(cb <= rb, s, -1e9)
                    m = jnp.max(s, axis=1, keepdims=True)
                    w = jnp.exp(s - m)
                    w = w / jnp.sum(w, axis=1, keepdims=True)
                    ctx_ref[h, r0:r0 + QC, :] = jnp.dot(
                        w, vh_ref[slot, :kn, :],
                        preferred_element_type=jnp.float32)
                chunk_copy(h, 1 if h % 2 == 0 else 3).start()

        @pl.when(my == 1)
        def _dev1():
            for h in EVENS:
                chunk_copy(h, 2).wait_recv()
                chunk_copy(h, 2).start()
            for h in ODDS:
                chunk_copy(h, 2).wait_recv()

        @pl.when(my == 2)
        def _dev2():
            for h in range(HQ):
                tgt = 3 if h % 2 == 0 else 1
                chunk_copy(h, tgt).wait_recv()
                chunk_copy(h, tgt).start()

        @pl.when(my == 3)
        def _dev3():
            for h in ODDS:
                chunk_copy(h, 2).wait_recv()
                chunk_copy(h, 2).start()
            for h in EVENS:
                chunk_copy(h, 2).wait_recv()

        for c in range(N_OC):
            r0 = c * OC
            acc = jnp.dot(ctx_ref[0, r0:r0 + OC, :], wo_ref[0:DH, :],
                          preferred_element_type=jnp.float32)
            for h in range(1, HQ):
                acc = acc + jnp.dot(
                    ctx_ref[h, r0:r0 + OC, :],
                    wo_ref[h * DH:(h + 1) * DH, :],
                    preferred_element_type=jnp.float32)
            oc_ref[...] = acc
            cp = pltpu.make_async_copy(
                oc_ref, out_ref.at[0, pl.ds(r0, OC)], dma_sem)
            cp.start()
            cp.wait()

        @pl.when(my == 0)
        def _drain0():
            for h in range(HQ):
                chunk_copy(h, 1 if h % 2 == 0 else 3).wait_send()

        @pl.when(my == 1)
        def _drain1():
            for h in EVENS:
                chunk_copy(h, 2).wait_send()

        @pl.when(my == 2)
        def _drain2():
            for h in range(HQ):
                chunk_copy(h, 3 if h % 2 == 0 else 1).wait_send()

        @pl.when(my == 3)
        def _drain3():
            for h in ODDS:
                chunk_copy(h, 2).wait_send()

    return pl.pallas_call(
        body,
        out_shape=jax.ShapeDtypeStruct((1, SQ, DM), jnp.float32),
        in_specs=[
            pl.BlockSpec(memory_space=pl.ANY),
            pl.BlockSpec(memory_space=pltpu.VMEM),
            pl.BlockSpec(memory_space=pl.ANY),
            pl.BlockSpec(memory_space=pl.ANY),
            pl.BlockSpec(memory_space=pltpu.VMEM),
        ],
        out_specs=pl.BlockSpec(memory_space=pl.ANY),
        scratch_shapes=[
            pltpu.VMEM((HQ, SQ, DH), jnp.float32),
            pltpu.VMEM((OC, DM), jnp.float32),
            pltpu.VMEM((2, SKV, DH), jnp.float32),
            pltpu.VMEM((2, SKV, DH), jnp.float32),
            pltpu.VMEM((2, SQ, DH), jnp.float32),
            pltpu.VMEM((OC, DM), jnp.float32),
            pltpu.SemaphoreType.DMA,
            pltpu.SemaphoreType.DMA((2,)),
            pltpu.SemaphoreType.DMA((2,)),
            pltpu.SemaphoreType.DMA((2,)),
            pltpu.SemaphoreType.DMA((HQ,)),
            pltpu.SemaphoreType.DMA((HQ,)),
        ],
    )(x, Wq, K_ext, V_ext, Wo)


# device time: 114000 ns/iter; 3.2325x vs baseline; 1.1180x over previous
import jax
import jax.numpy as jnp
from jax import lax
from jax.experimental import pallas as pl
from jax.experimental.pallas import tpu as pltpu

N_DEV = 4
SQ = 2048
SKV = 2048
HQ = 8
DH = 128
DM = HQ * DH
BLK = 64
SCALE = 0.08838834764831843
QC = 256
N_AQC = SQ // QC
OC = 512
N_OC = SQ // OC
HALF = SQ // 2

N_CH = 2 * HQ
SI_C14_D3, SI_C15_D1, SI_C12_D3, SI_C13_D1 = 16, 17, 18, 19
N_SEM = 20


def kernel(x, Wq, K_ext, V_ext, Wo):
    def body(x_ref, wq_ref, k_ref, v_ref, wo_ref, out_ref,
             ctx_ref, xc_ref, kh_ref, vh_ref, qh_ref, oc_ref,
             dma_sem, ksem, vsem, qsem, send_sems, recv_sems):
        my = lax.axis_index("i")

        def ch_copy(c, tgt, si):
            h, half = c // 2, c % 2
            reg = ctx_ref.at[h, pl.ds(half * HALF, HALF), :]
            return pltpu.make_async_remote_copy(
                src_ref=reg, dst_ref=reg,
                send_sem=send_sems.at[si], recv_sem=recv_sems.at[si],
                device_id=(tgt,), device_id_type=pl.DeviceIdType.MESH,
            )

        def kv_loads(h):
            slot = h % 2
            return (
                pltpu.make_async_copy(
                    k_ref.at[0, :, h, :], kh_ref.at[slot], ksem.at[slot]),
                pltpu.make_async_copy(
                    v_ref.at[0, :, h, :], vh_ref.at[slot], vsem.at[slot]),
            )

        def q_load(h):
            slot = h % 2
            return pltpu.make_async_copy(
                out_ref.at[0, :, pl.ds(h * DH, DH)], qh_ref.at[slot],
                qsem.at[slot])

        @pl.when(my == 0)
        def _attn():
            for op in kv_loads(0):
                op.start()
            for c in range(N_OC):
                r0 = c * OC
                cp = pltpu.make_async_copy(
                    x_ref.at[0, pl.ds(r0, OC)], xc_ref, dma_sem)
                cp.start()
                cp.wait()
                oc_ref[...] = jnp.dot(xc_ref[...], wq_ref[...],
                                      preferred_element_type=jnp.float32)
                cp2 = pltpu.make_async_copy(
                    oc_ref, out_ref.at[0, pl.ds(r0, OC)], dma_sem)
                cp2.start()
                cp2.wait()

            q_load(0).start()
            for h in range(HQ):
                if h + 1 < HQ:
                    for op in kv_loads(h + 1):
                        op.start()
                    q_load(h + 1).start()
                for op in kv_loads(h):
                    op.wait()
                q_load(h).wait()
                slot = h % 2
                for qc in range(N_AQC):
                    r0 = qc * QC
                    kn = r0 + QC
                    s = lax.dot_general(
                        qh_ref[slot, r0:r0 + QC, :], kh_ref[slot, :kn, :],
                        (((1,), (1,)), ((), ())),
                        preferred_element_type=jnp.float32,
                    ) * SCALE
                    rb = (r0 + lax.broadcasted_iota(
                        jnp.int32, (QC, 1), 0)) // BLK
                    cb = lax.broadcasted_iota(jnp.int32, (1, kn), 1) // BLK
                    s = jnp.where(cb <= rb, s, -1e9)
                    m = jnp.max(s, axis=1, keepdims=True)
                    w = jnp.exp(s - m)
                    w = w / jnp.sum(w, axis=1, keepdims=True)
                    ctx_ref[h, r0:r0 + QC, :] = jnp.dot(
                        w, vh_ref[slot, :kn, :],
                        preferred_element_type=jnp.float32)
                    if qc == N_AQC // 2 - 1:
                        ch_copy(2 * h, 1, 2 * h).start()
                        if h == 6:
                            ch_copy(12, 3, SI_C12_D3).start()
                        if h == 7:
                            ch_copy(14, 3, SI_C14_D3).start()
                ch_copy(2 * h + 1, 3, 2 * h + 1).start()
                if h == 6:
                    ch_copy(13, 1, SI_C13_D1).start()
                if h == 7:
                    ch_copy(15, 1, SI_C15_D1).start()

        @pl.when(my == 1)
        def _dev1():
            for h in range(HQ):
                ch_copy(2 * h, 2, 2 * h).wait_recv()
                ch_copy(2 * h, 2, 2 * h).start()
            for c in (1, 3, 5, 7, 9, 11):
                ch_copy(c, 2, c).wait_recv()
            ch_copy(13, 2, SI_C13_D1).wait_recv()
            ch_copy(15, 2, SI_C15_D1).wait_recv()

        @pl.when(my == 2)
        def _dev2():
            for h in range(6):
                ch_copy(2 * h, 3, 2 * h).wait_recv()
                ch_copy(2 * h, 3, 2 * h).start()
                ch_copy(2 * h + 1, 1, 2 * h + 1).wait_recv()
                ch_copy(2 * h + 1, 1, 2 * h + 1).start()
            for c in (12, 13, 14, 15):
                ch_copy(c, 1, c).wait_recv()

        @pl.when(my == 3)
        def _dev3():
            for h in range(HQ):
                ch_copy(2 * h + 1, 2, 2 * h + 1).wait_recv()
                ch_copy(2 * h + 1, 2, 2 * h + 1).start()
            for c in (0, 2, 4, 6, 8, 10):
                ch_copy(c, 2, c).wait_recv()
            ch_copy(12, 2, SI_C12_D3).wait_recv()
            ch_copy(14, 2, SI_C14_D3).wait_recv()

        for c in range(N_OC):
            r0 = c * OC
            acc = jnp.dot(ctx_ref[0, r0:r0 + OC, :], wo_ref[0:DH, :],
                          preferred_element_type=jnp.float32)
            for h in range(1, HQ):
                acc = acc + jnp.dot(
                    ctx_ref[h, r0:r0 + OC, :],
                    wo_ref[h * DH:(h + 1) * DH, :],
                    preferred_element_type=jnp.float32)
            oc_ref[...] = acc
            cp = pltpu.make_async_copy(
                oc_ref, out_ref.at[0, pl.ds(r0, OC)], dma_sem)
            cp.start()
            cp.wait()

        @pl.when(my == 0)
        def _drain0():
            for c in range(N_CH):
                ch_copy(c, 1 if c % 2 == 0 else 3, c).wait_send()
            ch_copy(14, 3, SI_C14_D3).wait_send()
            ch_copy(15, 1, SI_C15_D1).wait_send()
            ch_copy(12, 3, SI_C12_D3).wait_send()
            ch_copy(13, 1, SI_C13_D1).wait_send()

        @pl.when(my == 1)
        def _drain1():
            for h in range(HQ):
                ch_copy(2 * h, 2, 2 * h).wait_send()

        @pl.when(my == 2)
        def _drain2():
            for h in range(6):
                ch_copy(2 * h, 3, 2 * h).wait_send()
                ch_copy(2 * h + 1, 1, 2 * h + 1).wait_send()

        @pl.when(my == 3)
        def _drain3():
            for h in range(HQ):
                ch_copy(2 * h + 1, 2, 2 * h + 1).wait_send()

    return pl.pallas_call(
        body,
        out_shape=jax.ShapeDtypeStruct((1, SQ, DM), jnp.float32),
        in_specs=[
            pl.BlockSpec(memory_space=pl.ANY),
            pl.BlockSpec(memory_space=pltpu.VMEM),
            pl.BlockSpec(memory_space=pl.ANY),
            pl.BlockSpec(memory_space=pl.ANY),
            pl.BlockSpec(memory_space=pltpu.VMEM),
        ],
        out_specs=pl.BlockSpec(memory_space=pl.ANY),
        scratch_shapes=[
            pltpu.VMEM((HQ, SQ, DH), jnp.float32),
            pltpu.VMEM((OC, DM), jnp.float32),
            pltpu.VMEM((2, SKV, DH), jnp.float32),
            pltpu.VMEM((2, SKV, DH), jnp.float32),
            pltpu.VMEM((2, SQ, DH), jnp.float32),
            pltpu.VMEM((OC, DM), jnp.float32),
            pltpu.SemaphoreType.DMA,
            pltpu.SemaphoreType.DMA((2,)),
            pltpu.SemaphoreType.DMA((2,)),
            pltpu.SemaphoreType.DMA((2,)),
            pltpu.SemaphoreType.DMA((N_SEM,)),
            pltpu.SemaphoreType.DMA((N_SEM,)),
        ],
    )(x, Wq, K_ext, V_ext, Wo)


# device time: 103358 ns/iter; 3.5653x vs baseline; 1.1030x over previous
import jax
import jax.numpy as jnp
from jax import lax
from jax.experimental import pallas as pl
from jax.experimental.pallas import tpu as pltpu

N_DEV = 4
SQ = 2048
SKV = 2048
HQ = 8
DH = 128
DM = HQ * DH
BLK = 64
SCALE = 0.08838834764831843
QC = 256
N_AQC = SQ // QC
OC = 512
N_OC = SQ // OC
HALF = SQ // 2

N_CH = 2 * HQ
SI_C14_D3, SI_C15_D1, SI_C12_D3, SI_C13_D1 = 16, 17, 18, 19
N_SEM = 20


def kernel(x, Wq, K_ext, V_ext, Wo):
    def body(x_ref, wq_ref, k_ref, v_ref, wo_ref, out_ref,
             ctx_ref, xc_ref, kh_ref, vh_ref, qh_ref, oc_ref,
             dma_sem, ksem, vsem, qsem, send_sems, recv_sems):
        my = lax.axis_index("i")

        def ch_copy(c, tgt, si):
            h, half = c // 2, c % 2
            reg = ctx_ref.at[h, pl.ds(half * HALF, HALF), :]
            return pltpu.make_async_remote_copy(
                src_ref=reg, dst_ref=reg,
                send_sem=send_sems.at[si], recv_sem=recv_sems.at[si],
                device_id=(tgt,), device_id_type=pl.DeviceIdType.MESH,
            )

        def kv_loads(h):
            slot = h % 2
            return (
                pltpu.make_async_copy(
                    k_ref.at[0, :, h, :], kh_ref.at[slot], ksem.at[slot]),
                pltpu.make_async_copy(
                    v_ref.at[0, :, h, :], vh_ref.at[slot], vsem.at[slot]),
            )

        def q_load(h):
            slot = h % 2
            return pltpu.make_async_copy(
                out_ref.at[0, :, pl.ds(h * DH, DH)], qh_ref.at[slot],
                qsem.at[slot])

        @pl.when(my == 0)
        def _attn():
            rbl = lax.broadcasted_iota(jnp.int32, (QC, 1), 0) // BLK
            cbl = lax.broadcasted_iota(jnp.int32, (1, QC), 1) // BLK
            bias = jnp.where(cbl <= rbl, 0.0, -1e9).astype(jnp.float32)
            for op in kv_loads(0):
                op.start()
            for c in range(N_OC):
                r0 = c * OC
                cp = pltpu.make_async_copy(
                    x_ref.at[0, pl.ds(r0, OC)], xc_ref, dma_sem)
                cp.start()
                cp.wait()
                oc_ref[...] = jnp.dot(xc_ref[...], wq_ref[...],
                                      preferred_element_type=jnp.float32)
                cp2 = pltpu.make_async_copy(
                    oc_ref, out_ref.at[0, pl.ds(r0, OC)], dma_sem)
                cp2.start()
                cp2.wait()

            q_load(0).start()
            for h in range(HQ):
                if h + 1 < HQ:
                    for op in kv_loads(h + 1):
                        op.start()
                    q_load(h + 1).start()
                for op in kv_loads(h):
                    op.wait()
                q_load(h).wait()
                slot = h % 2
                qhs = qh_ref[slot] * SCALE
                for qc in range(N_AQC):
                    r0 = qc * QC
                    kn = r0 + QC
                    qck = qhs[r0:r0 + QC, :]
                    wd = jnp.exp(lax.dot_general(
                        qck, kh_ref[slot, r0:kn, :],
                        (((1,), (1,)), ((), ())),
                        preferred_element_type=jnp.float32,
                    ) + bias)
                    acc = jnp.dot(wd, vh_ref[slot, r0:kn, :],
                                  preferred_element_type=jnp.float32)
                    ssum = jnp.sum(wd, axis=1, keepdims=True)
                    if r0 > 0:
                        wl = jnp.exp(lax.dot_general(
                            qck, kh_ref[slot, :r0, :],
                            (((1,), (1,)), ((), ())),
                            preferred_element_type=jnp.float32,
                        ))
                        acc = acc + jnp.dot(
                            wl, vh_ref[slot, :r0, :],
                            preferred_element_type=jnp.float32)
                        ssum = ssum + jnp.sum(wl, axis=1, keepdims=True)
                    ctx_ref[h, r0:r0 + QC, :] = acc / ssum
                    if qc == N_AQC // 2 - 1:
                        ch_copy(2 * h, 1, 2 * h).start()
                        if h == 6:
                            ch_copy(12, 3, SI_C12_D3).start()
                        if h == 7:
                            ch_copy(14, 3, SI_C14_D3).start()
                ch_copy(2 * h + 1, 3, 2 * h + 1).start()
                if h == 6:
                    ch_copy(13, 1, SI_C13_D1).start()
                if h == 7:
                    ch_copy(15, 1, SI_C15_D1).start()

        @pl.when(my == 1)
        def _dev1():
            for h in range(HQ):
                ch_copy(2 * h, 2, 2 * h).wait_recv()
                ch_copy(2 * h, 2, 2 * h).start()
            for c in (1, 3, 5, 7, 9, 11):
                ch_copy(c, 2, c).wait_recv()
            ch_copy(13, 2, SI_C13_D1).wait_recv()
            ch_copy(15, 2, SI_C15_D1).wait_recv()

        @pl.when(my == 2)
        def _dev2():
            for h in range(6):
                ch_copy(2 * h, 3, 2 * h).wait_recv()
                ch_copy(2 * h, 3, 2 * h).start()
                ch_copy(2 * h + 1, 1, 2 * h + 1).wait_recv()
                ch_copy(2 * h + 1, 1, 2 * h + 1).start()
            for c in (12, 13, 14, 15):
                ch_copy(c, 1, c).wait_recv()

        @pl.when(my == 3)
        def _dev3():
            for h in range(HQ):
                ch_copy(2 * h + 1, 2, 2 * h + 1).wait_recv()
                ch_copy(2 * h + 1, 2, 2 * h + 1).start()
            for c in (0, 2, 4, 6, 8, 10):
                ch_copy(c, 2, c).wait_recv()
            ch_copy(12, 2, SI_C12_D3).wait_recv()
            ch_copy(14, 2, SI_C14_D3).wait_recv()

        for c in range(N_OC):
            r0 = c * OC
            acc = jnp.dot(ctx_ref[0, r0:r0 + OC, :], wo_ref[0:DH, :],
                          preferred_element_type=jnp.float32)
            for h in range(1, HQ):
                acc = acc + jnp.dot(
                    ctx_ref[h, r0:r0 + OC, :],
                    wo_ref[h * DH:(h + 1) * DH, :],
                    preferred_element_type=jnp.float32)
            oc_ref[...] = acc
            cp = pltpu.make_async_copy(
                oc_ref, out_ref.at[0, pl.ds(r0, OC)], dma_sem)
            cp.start()
            cp.wait()

        @pl.when(my == 0)
        def _drain0():
            for c in range(N_CH):
                ch_copy(c, 1 if c % 2 == 0 else 3, c).wait_send()
            ch_copy(14, 3, SI_C14_D3).wait_send()
            ch_copy(15, 1, SI_C15_D1).wait_send()
            ch_copy(12, 3, SI_C12_D3).wait_send()
            ch_copy(13, 1, SI_C13_D1).wait_send()

        @pl.when(my == 1)
        def _drain1():
            for h in range(HQ):
                ch_copy(2 * h, 2, 2 * h).wait_send()

        @pl.when(my == 2)
        def _drain2():
            for h in range(6):
                ch_copy(2 * h, 3, 2 * h).wait_send()
                ch_copy(2 * h + 1, 1, 2 * h + 1).wait_send()

        @pl.when(my == 3)
        def _drain3():
            for h in range(HQ):
                ch_copy(2 * h + 1, 2, 2 * h + 1).wait_send()

    return pl.pallas_call(
        body,
        out_shape=jax.ShapeDtypeStruct((1, SQ, DM), jnp.float32),
        in_specs=[
            pl.BlockSpec(memory_space=pl.ANY),
            pl.BlockSpec(memory_space=pltpu.VMEM),
            pl.BlockSpec(memory_space=pl.ANY),
            pl.BlockSpec(memory_space=pl.ANY),
            pl.BlockSpec(memory_space=pltpu.VMEM),
        ],
        out_specs=pl.BlockSpec(memory_space=pl.ANY),
        scratch_shapes=[
            pltpu.VMEM((HQ, SQ, DH), jnp.float32),
            pltpu.VMEM((OC, DM), jnp.float32),
            pltpu.VMEM((2, SKV, DH), jnp.float32),
            pltpu.VMEM((2, SKV, DH), jnp.float32),
            pltpu.VMEM((2, SQ, DH), jnp.float32),
            pltpu.VMEM((OC, DM), jnp.float32),
            pltpu.SemaphoreType.DMA,
            pltpu.SemaphoreType.DMA((2,)),
            pltpu.SemaphoreType.DMA((2,)),
            pltpu.SemaphoreType.DMA((2,)),
            pltpu.SemaphoreType.DMA((N_SEM,)),
            pltpu.SemaphoreType.DMA((N_SEM,)),
        ],
    )(x, Wq, K_ext, V_ext, Wo)


# device time: 82753 ns/iter; 4.4530x vs baseline; 1.2490x over previous
import jax
import jax.numpy as jnp
from jax import lax
from jax.experimental import pallas as pl
from jax.experimental.pallas import tpu as pltpu

N_DEV = 4
SQ = 2048
SKV = 2048
HQ = 8
DH = 128
DM = HQ * DH
BLK = 64
SCALE = 0.08838834764831843
QC = 256
N_AQC = SQ // QC
OC = 512
N_OC = SQ // OC
HALF = SQ // 2

N_CH = 2 * HQ
SI_C14_D3, SI_C15_D1, SI_C12_D3, SI_C13_D1 = 16, 17, 18, 19
N_SEM = 20


def kernel(x, Wq, K_ext, V_ext, Wo):
    def body(x_ref, wq_ref, k_ref, v_ref, wo_ref, out_ref,
             ctx_ref, acc_ref, xc_ref, kh_ref, vh_ref, qh_ref,
             dma_sem, ksem, vsem, qsem, send_sems, recv_sems):
        my = lax.axis_index("i")

        def ch_copy(c, tgt, si):
            h, half = c // 2, c % 2
            reg = ctx_ref.at[h, pl.ds(half * HALF, HALF), :]
            return pltpu.make_async_remote_copy(
                src_ref=reg, dst_ref=reg,
                send_sem=send_sems.at[si], recv_sem=recv_sems.at[si],
                device_id=(tgt,), device_id_type=pl.DeviceIdType.MESH,
            )

        def kv_loads(h):
            slot = h % 2
            return (
                pltpu.make_async_copy(
                    k_ref.at[0, :, h, :], kh_ref.at[slot], ksem.at[slot]),
                pltpu.make_async_copy(
                    v_ref.at[0, :, h, :], vh_ref.at[slot], vsem.at[slot]),
            )

        def q_load(h):
            slot = h % 2
            return pltpu.make_async_copy(
                out_ref.at[0, :, pl.ds(h * DH, DH)], qh_ref.at[slot],
                qsem.at[slot])

        def accum_chunk(c, first):
            h, half = c // 2, c % 2
            woh = wo_ref[h * DH:(h + 1) * DH, :]
            for p in range(HALF // OC):
                r0 = half * HALF + p * OC
                v = jnp.dot(ctx_ref[h, r0:r0 + OC, :], woh,
                            preferred_element_type=jnp.float32)
                if first:
                    acc_ref[r0:r0 + OC, :] = v
                else:
                    acc_ref[r0:r0 + OC, :] = acc_ref[r0:r0 + OC, :] + v

        @pl.when(my == 0)
        def _attn():
            rbl = lax.broadcasted_iota(jnp.int32, (QC, 1), 0) // BLK
            cbl = lax.broadcasted_iota(jnp.int32, (1, QC), 1) // BLK
            bias = jnp.where(cbl <= rbl, 0.0, -1e9).astype(jnp.float32)

            for op in kv_loads(0):
                op.start()
            for c in range(N_OC):
                r0 = c * OC
                cp = pltpu.make_async_copy(
                    x_ref.at[0, pl.ds(r0, OC)], xc_ref, dma_sem)
                cp.start()
                cp.wait()
                acc_ref[0:OC, :] = jnp.dot(
                    xc_ref[...], wq_ref[...],
                    preferred_element_type=jnp.float32)
                cp2 = pltpu.make_async_copy(
                    acc_ref.at[pl.ds(0, OC)], out_ref.at[0, pl.ds(r0, OC)],
                    dma_sem)
                cp2.start()
                cp2.wait()

            q_load(0).start()
            for h in range(HQ):
                if h + 1 < HQ:
                    for op in kv_loads(h + 1):
                        op.start()
                    q_load(h + 1).start()
                for op in kv_loads(h):
                    op.wait()
                q_load(h).wait()
                slot = h % 2
                qhs = qh_ref[slot] * SCALE
                for qc in range(N_AQC):
                    r0 = qc * QC
                    kn = r0 + QC
                    qck = qhs[r0:r0 + QC, :]
                    wd = jnp.exp(lax.dot_general(
                        qck, kh_ref[slot, r0:kn, :],
                        (((1,), (1,)), ((), ())),
                        preferred_element_type=jnp.float32,
                    ) + bias)
                    att = jnp.dot(wd, vh_ref[slot, r0:kn, :],
                                  preferred_element_type=jnp.float32)
                    ssum = jnp.sum(wd, axis=1, keepdims=True)
                    if r0 > 0:
                        wl = jnp.exp(lax.dot_general(
                            qck, kh_ref[slot, :r0, :],
                            (((1,), (1,)), ((), ())),
                            preferred_element_type=jnp.float32,
                        ))
                        att = att + jnp.dot(
                            wl, vh_ref[slot, :r0, :],
                            preferred_element_type=jnp.float32)
                        ssum = ssum + jnp.sum(wl, axis=1, keepdims=True)
                    ctx_ref[h, r0:r0 + QC, :] = (
                        att / ssum).astype(jnp.bfloat16)
                    if qc == N_AQC // 2 - 1:
                        ch_copy(2 * h, 1, 2 * h).start()
                        if h == 6:
                            ch_copy(12, 3, SI_C12_D3).start()
                        if h == 7:
                            ch_copy(14, 3, SI_C14_D3).start()
                ch_copy(2 * h + 1, 3, 2 * h + 1).start()
                if h == 6:
                    ch_copy(13, 1, SI_C13_D1).start()
                if h == 7:
                    ch_copy(15, 1, SI_C15_D1).start()
                accum_chunk(2 * h, h == 0)
                accum_chunk(2 * h + 1, h == 0)

        @pl.when(my == 1)
        def _dev1():
            for h in range(HQ):
                ch_copy(2 * h, 2, 2 * h).wait_recv()
                ch_copy(2 * h, 2, 2 * h).start()
                accum_chunk(2 * h, h == 0)
            for c in (1, 3, 5, 7, 9, 11):
                ch_copy(c, 2, c).wait_recv()
                accum_chunk(c, c == 1)
            ch_copy(13, 2, SI_C13_D1).wait_recv()
            accum_chunk(13, False)
            ch_copy(15, 2, SI_C15_D1).wait_recv()
            accum_chunk(15, False)

        @pl.when(my == 2)
        def _dev2():
            for h in range(6):
                ch_copy(2 * h, 3, 2 * h).wait_recv()
                ch_copy(2 * h, 3, 2 * h).start()
                ch_copy(2 * h + 1, 1, 2 * h + 1).wait_recv()
                ch_copy(2 * h + 1, 1, 2 * h + 1).start()
                accum_chunk(2 * h, h == 0)
                accum_chunk(2 * h + 1, h == 0)
            for c in (12, 13, 14, 15):
                ch_copy(c, 1, c).wait_recv()
                accum_chunk(c, False)

        @pl.when(my == 3)
        def _dev3():
            for h in range(HQ):
                ch_copy(2 * h + 1, 2, 2 * h + 1).wait_recv()
                ch_copy(2 * h + 1, 2, 2 * h + 1).start()
                accum_chunk(2 * h + 1, h == 0)
            for c in (0, 2, 4, 6, 8, 10):
                ch_copy(c, 2, c).wait_recv()
                accum_chunk(c, c == 0)
            ch_copy(12, 2, SI_C12_D3).wait_recv()
            accum_chunk(12, False)
            ch_copy(14, 2, SI_C14_D3).wait_recv()
            accum_chunk(14, False)

        cp = pltpu.make_async_copy(acc_ref, out_ref.at[0], dma_sem)
        cp.start()
        cp.wait()

        @pl.when(my == 0)
        def _drain0():
            for c in range(N_CH):
                ch_copy(c, 1 if c % 2 == 0 else 3, c).wait_send()
            ch_copy(14, 3, SI_C14_D3).wait_send()
            ch_copy(15, 1, SI_C15_D1).wait_send()
            ch_copy(12, 3, SI_C12_D3).wait_send()
            ch_copy(13, 1, SI_C13_D1).wait_send()

        @pl.when(my == 1)
        def _drain1():
            for h in range(HQ):
                ch_copy(2 * h, 2, 2 * h).wait_send()

        @pl.when(my == 2)
        def _drain2():
            for h in range(6):
                ch_copy(2 * h, 3, 2 * h).wait_send()
                ch_copy(2 * h + 1, 1, 2 * h + 1).wait_send()

        @pl.when(my == 3)
        def _drain3():
            for h in range(HQ):
                ch_copy(2 * h + 1, 2, 2 * h + 1).wait_send()

    return pl.pallas_call(
        body,
        out_shape=jax.ShapeDtypeStruct((1, SQ, DM), jnp.float32),
        in_specs=[
            pl.BlockSpec(memory_space=pl.ANY),
            pl.BlockSpec(memory_space=pltpu.VMEM),
            pl.BlockSpec(memory_space=pl.ANY),
            pl.BlockSpec(memory_space=pl.ANY),
            pl.BlockSpec(memory_space=pltpu.VMEM),
        ],
        out_specs=pl.BlockSpec(memory_space=pl.ANY),
        scratch_shapes=[
            pltpu.VMEM((HQ, SQ, DH), jnp.bfloat16),
            pltpu.VMEM((SQ, DM), jnp.float32),
            pltpu.VMEM((OC, DM), jnp.float32),
            pltpu.VMEM((2, SKV, DH), jnp.float32),
            pltpu.VMEM((2, SKV, DH), jnp.float32),
            pltpu.VMEM((2, SQ, DH), jnp.float32),
            pltpu.SemaphoreType.DMA,
            pltpu.SemaphoreType.DMA((2,)),
            pltpu.SemaphoreType.DMA((2,)),
            pltpu.SemaphoreType.DMA((2,)),
            pltpu.SemaphoreType.DMA((N_SEM,)),
            pltpu.SemaphoreType.DMA((N_SEM,)),
        ],
    )(x, Wq, K_ext, V_ext, Wo)


# device time: 81928 ns/iter; 4.4979x vs baseline; 1.0101x over previous
import jax
import jax.numpy as jnp
from jax import lax
from jax.experimental import pallas as pl
from jax.experimental.pallas import tpu as pltpu

N_DEV = 4
SQ = 2048
SKV = 2048
HQ = 8
DH = 128
DM = HQ * DH
BLK = 64
SCALE = 0.08838834764831843
QC = 256
N_AQC = SQ // QC
OC = 512
N_OC = SQ // OC
HALF = SQ // 2

N_CH = 2 * HQ
SI_C14_D3, SI_C15_D1, SI_C12_D3, SI_C13_D1 = 16, 17, 18, 19
N_SEM = 20


def kernel(x, Wq, K_ext, V_ext, Wo):
    def body(x_ref, wq_ref, k_ref, v_ref, wo_ref, out_ref,
             ctx_ref, acc_ref, kh_ref, vh_ref,
             dma_sem, ksem, vsem, send_sems, recv_sems):
        my = lax.axis_index("i")

        def ch_copy(c, tgt, si):
            h, half = c // 2, c % 2
            reg = ctx_ref.at[h, pl.ds(half * HALF, HALF), :]
            return pltpu.make_async_remote_copy(
                src_ref=reg, dst_ref=reg,
                send_sem=send_sems.at[si], recv_sem=recv_sems.at[si],
                device_id=(tgt,), device_id_type=pl.DeviceIdType.MESH,
            )

        def kv_loads(h):
            slot = h % 2
            return (
                pltpu.make_async_copy(
                    k_ref.at[0, :, h, :], kh_ref.at[slot], ksem.at[slot]),
                pltpu.make_async_copy(
                    v_ref.at[0, :, h, :], vh_ref.at[slot], vsem.at[slot]),
            )

        def store_half(half):
            return pltpu.make_async_copy(
                acc_ref.at[pl.ds(half * HALF, HALF)],
                out_ref.at[0, pl.ds(half * HALF, HALF)],
                dma_sem.at[half])

        def accum_chunk(c, first):
            h, half = c // 2, c % 2
            woh = wo_ref[h * DH:(h + 1) * DH, :]
            for p in range(HALF // OC):
                r0 = half * HALF + p * OC
                v = jnp.dot(ctx_ref[h, r0:r0 + OC, :], woh,
                            preferred_element_type=jnp.float32)
                if first:
                    acc_ref[r0:r0 + OC, :] = v
                else:
                    acc_ref[r0:r0 + OC, :] = acc_ref[r0:r0 + OC, :] + v

        @pl.when(my == 0)
        def _attn():
            rbl = lax.broadcasted_iota(jnp.int32, (QC, 1), 0) // BLK
            cbl = lax.broadcasted_iota(jnp.int32, (1, QC), 1) // BLK
            bias = jnp.where(cbl <= rbl, 0.0, -1e9).astype(jnp.float32)

            for op in kv_loads(0):
                op.start()
            cpx = pltpu.make_async_copy(
                x_ref.at[0], acc_ref, dma_sem.at[0])
            cpx.start()
            cpx.wait()
            for h in range(HQ):
                ctx_ref[h, :, :] = (jnp.dot(
                    acc_ref[...], wq_ref[:, h * DH:(h + 1) * DH],
                    preferred_element_type=jnp.float32,
                ) * SCALE).astype(jnp.bfloat16)

            for h in range(HQ):
                if h + 1 < HQ:
                    for op in kv_loads(h + 1):
                        op.start()
                for op in kv_loads(h):
                    op.wait()
                slot = h % 2
                qhs = ctx_ref[h].astype(jnp.float32)
                for qc in range(N_AQC):
                    r0 = qc * QC
                    kn = r0 + QC
                    qck = qhs[r0:r0 + QC, :]
                    wd = jnp.exp(lax.dot_general(
                        qck, kh_ref[slot, r0:kn, :],
                        (((1,), (1,)), ((), ())),
                        preferred_element_type=jnp.float32,
                    ) + bias)
                    att = jnp.dot(wd, vh_ref[slot, r0:kn, :],
                                  preferred_element_type=jnp.float32)
                    ssum = jnp.sum(wd, axis=1, keepdims=True)
                    if r0 > 0:
                        wl = jnp.exp(lax.dot_general(
                            qck, kh_ref[slot, :r0, :],
                            (((1,), (1,)), ((), ())),
                            preferred_element_type=jnp.float32,
                        ))
                        att = att + jnp.dot(
                            wl, vh_ref[slot, :r0, :],
                            preferred_element_type=jnp.float32)
                        ssum = ssum + jnp.sum(wl, axis=1, keepdims=True)
                    ctx_ref[h, r0:r0 + QC, :] = (
                        att / ssum).astype(jnp.bfloat16)
                    if qc == N_AQC // 2 - 1:
                        ch_copy(2 * h, 1, 2 * h).start()
                        if h == 6:
                            ch_copy(12, 3, SI_C12_D3).start()
                        if h == 7:
                            ch_copy(14, 3, SI_C14_D3).start()
                ch_copy(2 * h + 1, 3, 2 * h + 1).start()
                if h == 6:
                    ch_copy(13, 1, SI_C13_D1).start()
                if h == 7:
                    ch_copy(15, 1, SI_C15_D1).start()
                accum_chunk(2 * h, h == 0)
                accum_chunk(2 * h + 1, h == 0)
            store_half(0).start()
            store_half(1).start()

        @pl.when(my == 1)
        def _dev1():
            for h in range(HQ):
                ch_copy(2 * h, 2, 2 * h).wait_recv()
                ch_copy(2 * h, 2, 2 * h).start()
                accum_chunk(2 * h, h == 0)
            store_half(0).start()
            for c in (1, 3, 5, 7, 9, 11):
                ch_copy(c, 2, c).wait_recv()
                accum_chunk(c, c == 1)
            ch_copy(13, 2, SI_C13_D1).wait_recv()
            accum_chunk(13, False)
            ch_copy(15, 2, SI_C15_D1).wait_recv()
            accum_chunk(15, False)
            store_half(1).start()

        @pl.when(my == 2)
        def _dev2():
            for h in range(6):
                ch_copy(2 * h, 3, 2 * h).wait_recv()
                ch_copy(2 * h, 3, 2 * h).start()
                ch_copy(2 * h + 1, 1, 2 * h + 1).wait_recv()
                ch_copy(2 * h + 1, 1, 2 * h + 1).start()
                accum_chunk(2 * h, h == 0)
                accum_chunk(2 * h + 1, h == 0)
            for c in (12, 13, 14, 15):
                ch_copy(c, 1, c).wait_recv()
                accum_chunk(c, False)
                if c == 14:
                    store_half(0).start()
            store_half(1).start()

        @pl.when(my == 3)
        def _dev3():
            for h in range(HQ):
                ch_copy(2 * h + 1, 2, 2 * h + 1).wait_recv()
                ch_copy(2 * h + 1, 2, 2 * h + 1).start()
                accum_chunk(2 * h + 1, h == 0)
            store_half(1).start()
            for c in (0, 2, 4, 6, 8, 10):
                ch_copy(c, 2, c).wait_recv()
                accum_chunk(c, c == 0)
            ch_copy(12, 2, SI_C12_D3).wait_recv()
            accum_chunk(12, False)
            ch_copy(14, 2, SI_C14_D3).wait_recv()
            accum_chunk(14, False)
            store_half(0).start()

        store_half(0).wait()
        store_half(1).wait()

        @pl.when(my == 0)
        def _drain0():
            for c in range(N_CH):
                ch_copy(c, 1 if c % 2 == 0 else 3, c).wait_send()
            ch_copy(14, 3, SI_C14_D3).wait_send()
            ch_copy(15, 1, SI_C15_D1).wait_send()
            ch_copy(12, 3, SI_C12_D3).wait_send()
            ch_copy(13, 1, SI_C13_D1).wait_send()

        @pl.when(my == 1)
        def _drain1():
            for h in range(HQ):
                ch_copy(2 * h, 2, 2 * h).wait_send()

        @pl.when(my == 2)
        def _drain2():
            for h in range(6):
                ch_copy(2 * h, 3, 2 * h).wait_send()
                ch_copy(2 * h + 1, 1, 2 * h + 1).wait_send()

        @pl.when(my == 3)
        def _drain3():
            for h in range(HQ):
                ch_copy(2 * h + 1, 2, 2 * h + 1).wait_send()

    return pl.pallas_call(
        body,
        out_shape=jax.ShapeDtypeStruct((1, SQ, DM), jnp.float32),
        in_specs=[
            pl.BlockSpec(memory_space=pl.ANY),
            pl.BlockSpec(memory_space=pltpu.VMEM),
            pl.BlockSpec(memory_space=pl.ANY),
            pl.BlockSpec(memory_space=pl.ANY),
            pl.BlockSpec(memory_space=pltpu.VMEM),
        ],
        out_specs=pl.BlockSpec(memory_space=pl.ANY),
        scratch_shapes=[
            pltpu.VMEM((HQ, SQ, DH), jnp.bfloat16),
            pltpu.VMEM((SQ, DM), jnp.float32),
            pltpu.VMEM((2, SKV, DH), jnp.float32),
            pltpu.VMEM((2, SKV, DH), jnp.float32),
            pltpu.SemaphoreType.DMA((2,)),
            pltpu.SemaphoreType.DMA((2,)),
            pltpu.SemaphoreType.DMA((2,)),
            pltpu.SemaphoreType.DMA((N_SEM,)),
            pltpu.SemaphoreType.DMA((N_SEM,)),
        ],
    )(x, Wq, K_ext, V_ext, Wo)


# device time: 77388 ns/iter; 4.7617x vs baseline; 1.0587x over previous
import jax
import jax.numpy as jnp
from jax import lax
from jax.experimental import pallas as pl
from jax.experimental.pallas import tpu as pltpu

N_DEV = 4
SQ = 2048
SKV = 2048
HQ = 8
DH = 128
DM = HQ * DH
BLK = 64
SCALE = 0.08838834764831843
QC = 256
N_AQC = SQ // QC
OC = 512
N_OC = SQ // OC
HALF = SQ // 2

N_CH = 2 * HQ
SI_C14_D3, SI_C15_D1, SI_C12_D3, SI_C13_D1 = 16, 17, 18, 19
N_SEM = 20


def kernel(x, Wq, K_ext, V_ext, Wo):
    def body(x_ref, wq_ref, k_ref, v_ref, wo_ref, out_ref,
             ctx_ref, acc_ref, kh_ref, vh_ref,
             dma_sem, ksem, vsem, send_sems, recv_sems):
        my = lax.axis_index("i")

        def ch_copy(c, tgt, si):
            h, half = c // 2, c % 2
            reg = ctx_ref.at[h, pl.ds(half * HALF, HALF), :]
            return pltpu.make_async_remote_copy(
                src_ref=reg, dst_ref=reg,
                send_sem=send_sems.at[si], recv_sem=recv_sems.at[si],
                device_id=(tgt,), device_id_type=pl.DeviceIdType.MESH,
            )

        def kv_loads(h):
            slot = h % 2
            return (
                pltpu.make_async_copy(
                    k_ref.at[0, :, h, :], kh_ref.at[slot], ksem.at[slot]),
                pltpu.make_async_copy(
                    v_ref.at[0, :, h, :], vh_ref.at[slot], vsem.at[slot]),
            )

        def store_half(half):
            return pltpu.make_async_copy(
                acc_ref.at[pl.ds(half * HALF, HALF)],
                out_ref.at[0, pl.ds(half * HALF, HALF)],
                dma_sem.at[half])

        def accum_chunk(c, first):
            h, half = c // 2, c % 2
            woh = wo_ref[h * DH:(h + 1) * DH, :]
            for p in range(HALF // OC):
                r0 = half * HALF + p * OC
                v = jnp.dot(ctx_ref[h, r0:r0 + OC, :], woh,
                            preferred_element_type=jnp.float32)
                if first:
                    acc_ref[r0:r0 + OC, :] = v
                else:
                    acc_ref[r0:r0 + OC, :] = acc_ref[r0:r0 + OC, :] + v

        @pl.when(my == 0)
        def _attn():
            rbl = lax.broadcasted_iota(jnp.int32, (QC, 1), 0) // BLK
            cbl = lax.broadcasted_iota(jnp.int32, (1, QC), 1) // BLK
            bias = jnp.where(cbl <= rbl, 0.0, -1e9).astype(jnp.float32)

            for op in kv_loads(0):
                op.start()
            cpx = pltpu.make_async_copy(
                x_ref.at[0], acc_ref, dma_sem.at[0])
            cpx.start()
            cpx.wait()
            for c in range(N_OC):
                r0 = c * OC
                qf = jnp.dot(acc_ref[r0:r0 + OC, :], wq_ref[...],
                             preferred_element_type=jnp.float32) * SCALE
                for h in range(HQ):
                    ctx_ref[h, r0:r0 + OC, :] = (
                        qf[:, h * DH:(h + 1) * DH]).astype(jnp.bfloat16)

            for h in range(HQ):
                if h + 1 < HQ:
                    for op in kv_loads(h + 1):
                        op.start()
                for op in kv_loads(h):
                    op.wait()
                slot = h % 2
                qhs = ctx_ref[h].astype(jnp.float32)
                for qc in range(N_AQC):
                    r0 = qc * QC
                    kn = r0 + QC
                    qck = qhs[r0:r0 + QC, :]
                    wd = jnp.exp(lax.dot_general(
                        qck, kh_ref[slot, r0:kn, :],
                        (((1,), (1,)), ((), ())),
                        preferred_element_type=jnp.float32,
                    ) + bias)
                    att = jnp.dot(wd, vh_ref[slot, r0:kn, :],
                                  preferred_element_type=jnp.float32)
                    ssum = jnp.sum(wd, axis=1, keepdims=True)
                    if r0 > 0:
                        wl = jnp.exp(lax.dot_general(
                            qck, kh_ref[slot, :r0, :],
                            (((1,), (1,)), ((), ())),
                            preferred_element_type=jnp.float32,
                        ))
                        att = att + jnp.dot(
                            wl, vh_ref[slot, :r0, :],
                            preferred_element_type=jnp.float32)
                        ssum = ssum + jnp.sum(wl, axis=1, keepdims=True)
                    ctx_ref[h, r0:r0 + QC, :] = (
                        att / ssum).astype(jnp.bfloat16)
                    if qc == N_AQC // 2 - 1:
                        ch_copy(2 * h, 1, 2 * h).start()
                        if h == 6:
                            ch_copy(12, 3, SI_C12_D3).start()
                        if h == 7:
                            ch_copy(14, 3, SI_C14_D3).start()
                ch_copy(2 * h + 1, 3, 2 * h + 1).start()
                if h == 6:
                    ch_copy(13, 1, SI_C13_D1).start()
                if h == 7:
                    ch_copy(15, 1, SI_C15_D1).start()
                accum_chunk(2 * h, h == 0)
                accum_chunk(2 * h + 1, h == 0)
            store_half(0).start()
            store_half(1).start()

        @pl.when(my == 1)
        def _dev1():
            for h in range(HQ):
                ch_copy(2 * h, 2, 2 * h).wait_recv()
                ch_copy(2 * h, 2, 2 * h).start()
                accum_chunk(2 * h, h == 0)
            store_half(0).start()
            for c in (1, 3, 5, 7, 9, 11):
                ch_copy(c, 2, c).wait_recv()
                accum_chunk(c, c == 1)
            ch_copy(13, 2, SI_C13_D1).wait_recv()
            accum_chunk(13, False)
            ch_copy(15, 2, SI_C15_D1).wait_recv()
            accum_chunk(15, False)
            store_half(1).start()

        @pl.when(my == 2)
        def _dev2():
            for h in range(6):
                ch_copy(2 * h, 3, 2 * h).wait_recv()
                ch_copy(2 * h, 3, 2 * h).start()
                ch_copy(2 * h + 1, 1, 2 * h + 1).wait_recv()
                ch_copy(2 * h + 1, 1, 2 * h + 1).start()
                accum_chunk(2 * h, h == 0)
                accum_chunk(2 * h + 1, h == 0)
            for c in (12, 13, 14, 15):
                ch_copy(c, 1, c).wait_recv()
                accum_chunk(c, False)
                if c == 14:
                    store_half(0).start()
            store_half(1).start()

        @pl.when(my == 3)
        def _dev3():
            for h in range(HQ):
                ch_copy(2 * h + 1, 2, 2 * h + 1).wait_recv()
                ch_copy(2 * h + 1, 2, 2 * h + 1).start()
                accum_chunk(2 * h + 1, h == 0)
            store_half(1).start()
            for c in (0, 2, 4, 6, 8, 10):
                ch_copy(c, 2, c).wait_recv()
                accum_chunk(c, c == 0)
            ch_copy(12, 2, SI_C12_D3).wait_recv()
            accum_chunk(12, False)
            ch_copy(14, 2, SI_C14_D3).wait_recv()
            accum_chunk(14, False)
            store_half(0).start()

        store_half(0).wait()
        store_half(1).wait()

        @pl.when(my == 0)
        def _drain0():
            for c in range(N_CH):
                ch_copy(c, 1 if c % 2 == 0 else 3, c).wait_send()
            ch_copy(14, 3, SI_C14_D3).wait_send()
            ch_copy(15, 1, SI_C15_D1).wait_send()
            ch_copy(12, 3, SI_C12_D3).wait_send()
            ch_copy(13, 1, SI_C13_D1).wait_send()

        @pl.when(my == 1)
        def _drain1():
            for h in range(HQ):
                ch_copy(2 * h, 2, 2 * h).wait_send()

        @pl.when(my == 2)
        def _drain2():
            for h in range(6):
                ch_copy(2 * h, 3, 2 * h).wait_send()
                ch_copy(2 * h + 1, 1, 2 * h + 1).wait_send()

        @pl.when(my == 3)
        def _drain3():
            for h in range(HQ):
                ch_copy(2 * h + 1, 2, 2 * h + 1).wait_send()

    return pl.pallas_call(
        body,
        out_shape=jax.ShapeDtypeStruct((1, SQ, DM), jnp.float32),
        in_specs=[
            pl.BlockSpec(memory_space=pl.ANY),
            pl.BlockSpec(memory_space=pltpu.VMEM),
            pl.BlockSpec(memory_space=pl.ANY),
            pl.BlockSpec(memory_space=pl.ANY),
            pl.BlockSpec(memory_space=pltpu.VMEM),
        ],
        out_specs=pl.BlockSpec(memory_space=pl.ANY),
        scratch_shapes=[
            pltpu.VMEM((HQ, SQ, DH), jnp.bfloat16),
            pltpu.VMEM((SQ, DM), jnp.float32),
            pltpu.VMEM((2, SKV, DH), jnp.float32),
            pltpu.VMEM((2, SKV, DH), jnp.float32),
            pltpu.SemaphoreType.DMA((2,)),
            pltpu.SemaphoreType.DMA((2,)),
            pltpu.SemaphoreType.DMA((2,)),
            pltpu.SemaphoreType.DMA((N_SEM,)),
            pltpu.SemaphoreType.DMA((N_SEM,)),
        ],
    )(x, Wq, K_ext, V_ext, Wo)
